# baseline (speedup 1.0000x reference)
import jax
import jax.numpy as jnp
from jax import lax
from jax.experimental import pallas as pl
from jax.experimental.pallas import tpu as pltpu

B, S, D_MODEL, Q_RANK, H, DH, TOP_K = 1, 2048, 2048, 1536, 8, 128, 512
BQ = 256


def _lane_sum(x):
    acc = pltpu.roll(x, 120, 1) + x
    for g in range(2, 16):
        acc = pltpu.roll(x, 128 - 8 * g, 1) + acc
    for d in (4, 2, 1):
        acc = pltpu.roll(acc, 128 - d, 1) + acc
    return acc[:, 0:1]


def _chunked_mm_t(a_ref, b_ref, k_total):
    acc = None
    for c0 in range(0, k_total, 256):
        p = lax.dot_general(a_ref[:, c0:c0 + 256], b_ref[:, c0:c0 + 256],
                            (((1,), (1,)), ((), ())),
                            preferred_element_type=jnp.float32)
        acc = p if acc is None else acc + p
    return acc


def _kw_kernel(x_ref, wk_ref, g_ref, b_ref, ww_ref, k_ref, w_ref):
    xk = jnp.dot(x_ref[...], wk_ref[...].T, preferred_element_type=jnp.float32)
    m = _lane_sum(xk) * jnp.float32(0.0078125)
    dev = xk - m
    v = _lane_sum(dev * dev) * jnp.float32(0.0078125)
    sv = v + jnp.float32(1e-5)
    sq = sv * lax.rsqrt(sv)
    k_ref[...] = (dev * (jnp.float32(1.0) / sq)) * g_ref[...] + b_ref[...]
    w = jnp.dot(x_ref[...], ww_ref[...].T, preferred_element_type=jnp.float32)
    w_ref[...] = w * (H ** -0.5)


def _q_kernel(q_in_ref, wq_ref, q_ref):
    q_ref[...] = _chunked_mm_t(q_in_ref, wq_ref, Q_RANK)


def _scores_topk_kernel(q_ref, w_ref, k_ref, o_ref):
    scale = DH ** -0.5
    w = w_ref[...]
    acc = jnp.zeros((BQ, S), jnp.float32)
    for h in range(H):
        sh = lax.dot_general(q_ref[:, h * DH:(h + 1) * DH], k_ref[...],
                             (((1,), (1,)), ((), ())),
                             preferred_element_type=jnp.float32)
        acc = acc + sh * (w[:, h:h + 1] * scale)
    v = acc
    idx = lax.broadcasted_iota(jnp.int32, (BQ, S), 1)
    iota = lax.broadcasted_iota(jnp.int32, (1, S), 1)
    m = 2
    while m <= S:
        d = m // 2
        while d >= 1:
            desc = (iota & m) == 0
            bit = (iota & d) != 0
            take_hi = desc != bit
            pv = jnp.where(bit, pltpu.roll(v, d, 1), pltpu.roll(v, S - d, 1))
            pi = jnp.where(bit, pltpu.roll(idx, d, 1), pltpu.roll(idx, S - d, 1))
            dom = (pv > v) | ((pv == v) & (pi < idx))
            use_p = dom == take_hi
            v = jnp.where(use_p, pv, v)
            idx = jnp.where(use_p, pi, idx)
            d //= 2
        m *= 2
    o_ref[...] = idx[:, :TOP_K]


def kernel(x, q_input, W_q, W_k, ln_g, ln_b, W_w):
    x2 = x.reshape(S, D_MODEL)
    q2 = q_input.reshape(S, Q_RANK)
    k, w = pl.pallas_call(
        _kw_kernel,
        out_shape=(jax.ShapeDtypeStruct((S, DH), jnp.float32),
                   jax.ShapeDtypeStruct((S, H), jnp.float32)),
    )(x2, W_k, ln_g, ln_b, W_w)
    q = pl.pallas_call(
        _q_kernel,
        out_shape=jax.ShapeDtypeStruct((S, H * DH), jnp.float32),
    )(q2, W_q)
    top_idx = pl.pallas_call(
        _scores_topk_kernel,
        grid=(S // BQ,),
        in_specs=[
            pl.BlockSpec((BQ, H * DH), lambda i: (i, 0)),
            pl.BlockSpec((BQ, H), lambda i: (i, 0)),
            pl.BlockSpec((S, DH), lambda i: (0, 0)),
        ],
        out_specs=pl.BlockSpec((BQ, TOP_K), lambda i: (i, 0)),
        out_shape=jax.ShapeDtypeStruct((S, TOP_K), jnp.int32),
    )(q, w, k)
    return top_idx.reshape(B, S, TOP_K)
